# baseline (speedup 1.0000x reference)
import jax
import jax.numpy as jnp
from jax import lax
from jax.experimental import pallas as pl
from jax.experimental.pallas import tpu as pltpu
from jax.experimental.pallas import tpu_sc as plsc

N_NODES = 10000
D_FEAT = 256
NUM_PATHS = 50000
PATH_LEN = 4

TOTAL_ROWS = NUM_PATHS * PATH_LEN
GCHUNK = 80
GPU_ = 2
UNIT = GCHUNK * GPU_
NUM_UNITS = TOTAL_ROWS // UNIT
NUM_WORKERS = 32
BASE_UNITS = NUM_UNITS // NUM_WORKERS
EXTRA = NUM_UNITS - BASE_UNITS * NUM_WORKERS
MAX_UNITS_W = BASE_UNITS + 1
NBUF = 3
STEPS = (MAX_UNITS_W + NBUF - 1) // NBUF
IDX_PAD = ((NUM_WORKERS - 1) * BASE_UNITS + EXTRA + MAX_UNITS_W) * UNIT


def _gather_body(idx_hbm, x_hbm, out_hbm, idx_v, rows_v, gsem, ssem):
    nc = jnp.int32(2)
    wid = lax.axis_index("s") * nc + lax.axis_index("c")
    count = jnp.int32(BASE_UNITS) + jnp.where(wid < EXTRA, 1, 0).astype(jnp.int32)
    start = jnp.int32(BASE_UNITS) * wid + jnp.minimum(wid, jnp.int32(EXTRA))
    pltpu.sync_copy(
        idx_hbm.at[pl.ds(start * jnp.int32(UNIT), MAX_UNITS_W * UNIT)], idx_v)

    def gather_args(u, b, g):
        bi = jnp.int32(b)
        off = u * jnp.int32(UNIT) + jnp.int32(g * GCHUNK)
        idx_c = idx_v.at[pl.ds(off, GCHUNK)]
        dst = rows_v.at[bi, pl.ds(g * GCHUNK, GCHUNK)]
        return x_hbm.at[idx_c], dst, gsem.at[bi]

    def store_args(u, b):
        bi = jnp.int32(b)
        row0 = (start + u) * jnp.int32(UNIT)
        return rows_v.at[bi], out_hbm.at[pl.ds(row0, UNIT)], ssem.at[bi]

    def gather(u, b):
        for g in range(GPU_):
            pltpu.async_copy(*gather_args(u, b, g))

    def gather_wait(u, b):
        for g in range(GPU_):
            pltpu.make_async_copy(*gather_args(u, b, g)).wait()

    def store(u, b):
        pltpu.async_copy(*store_args(u, b))

    def store_wait(u, b):
        pltpu.make_async_copy(*store_args(u, b)).wait()

    for b in range(NBUF):
        gather(jnp.int32(b), b)

    @pl.loop(jnp.int32(0), jnp.int32(STEPS))
    def step(s):
        ubase = s * jnp.int32(NBUF)
        for b in range(NBUF):
            u = ubase + jnp.int32(b)

            @pl.when(u < count)
            def _():
                gather_wait(u, b)
                store(u, b)

        for b in range(NBUF):
            u = ubase + jnp.int32(b)
            un = u + jnp.int32(NBUF)

            @pl.when(u < count)
            def _():
                store_wait(u, b)

            @pl.when(un < count)
            def _():
                gather(un, b)


@jax.jit
def _sc_gather(idx_flat, x):
    mesh = plsc.VectorSubcoreMesh(core_axis_name="c", subcore_axis_name="s")
    f = pl.kernel(
        _gather_body,
        mesh=mesh,
        out_type=jax.ShapeDtypeStruct((TOTAL_ROWS, D_FEAT), jnp.float32),
        scratch_types=[
            pltpu.VMEM((MAX_UNITS_W * UNIT,), jnp.int32),
            pltpu.VMEM((NBUF, UNIT, D_FEAT), jnp.float32),
            pltpu.SemaphoreType.DMA((NBUF,)),
            pltpu.SemaphoreType.DMA((NBUF,)),
        ],
    )
    return f(idx_flat, x)


def kernel(path_index_without_target, x, att):
    del att
    idx = path_index_without_target.T.reshape(-1).astype(jnp.int32)
    idx = jnp.pad(idx, (0, IDX_PAD - TOTAL_ROWS))
    out = _sc_gather(idx, x.astype(jnp.float32))
    return out.reshape(PATH_LEN, NUM_PATHS, D_FEAT)


# speedup vs baseline: 1.0281x; 1.0281x over previous
import jax
import jax.numpy as jnp
from jax import lax
from jax.experimental import pallas as pl
from jax.experimental.pallas import tpu as pltpu
from jax.experimental.pallas import tpu_sc as plsc

N_NODES = 10000
D_FEAT = 256
NUM_PATHS = 50000
PATH_LEN = 4

TOTAL_ROWS = NUM_PATHS * PATH_LEN
CHUNK = 80
NUM_CHUNKS = TOTAL_ROWS // CHUNK
NUM_WORKERS = 32
BASE_CHUNKS = NUM_CHUNKS // NUM_WORKERS
EXTRA = NUM_CHUNKS - BASE_CHUNKS * NUM_WORKERS
MAX_CHUNKS_W = BASE_CHUNKS + 1
NBUF = 5
SDELAY = 2
STEPS = (MAX_CHUNKS_W + SDELAY + NBUF - 1) // NBUF
IDX_PAD = ((NUM_WORKERS - 1) * BASE_CHUNKS + EXTRA + MAX_CHUNKS_W) * CHUNK


def _gather_body(idx_hbm, x_hbm, out_hbm, idx_v, rows_v, gsem, ssem):
    nc = jnp.int32(2)
    wid = lax.axis_index("s") * nc + lax.axis_index("c")
    count = jnp.int32(BASE_CHUNKS) + jnp.where(wid < EXTRA, 1, 0).astype(jnp.int32)
    start = jnp.int32(BASE_CHUNKS) * wid + jnp.minimum(wid, jnp.int32(EXTRA))
    pltpu.sync_copy(
        idx_hbm.at[pl.ds(start * jnp.int32(CHUNK), MAX_CHUNKS_W * CHUNK)], idx_v)

    def gather_args(j, b):
        bi = jnp.int32(b)
        idx_c = idx_v.at[pl.ds(j * jnp.int32(CHUNK), CHUNK)]
        return x_hbm.at[idx_c], rows_v.at[bi], gsem.at[bi]

    def store_args(j, b):
        bi = jnp.int32(b)
        row0 = (start + j) * jnp.int32(CHUNK)
        return rows_v.at[bi], out_hbm.at[pl.ds(row0, CHUNK)], ssem.at[bi]

    def gather(j, b):
        pltpu.async_copy(*gather_args(j, b))

    def gather_wait(j, b):
        pltpu.make_async_copy(*gather_args(j, b)).wait()

    def store(j, b):
        pltpu.async_copy(*store_args(j, b))

    def store_wait(j, b):
        pltpu.make_async_copy(*store_args(j, b)).wait()

    for b in range(NBUF):
        gather(jnp.int32(b), b)

    @pl.loop(jnp.int32(0), jnp.int32(STEPS))
    def step(s):
        jbase = s * jnp.int32(NBUF)
        for b in range(NBUF):
            j = jbase + jnp.int32(b)

            @pl.when(j < count)
            def _():
                gather_wait(j, b % NBUF)
                store(j, b % NBUF)

            jp = j - jnp.int32(SDELAY)
            bp = (b - SDELAY) % NBUF
            jn = jp + jnp.int32(NBUF)

            @pl.when((jp >= 0) & (jp < count))
            def _():
                store_wait(jp, bp)

            @pl.when((jn >= jnp.int32(NBUF)) & (jn < count))
            def _():
                gather(jn, bp)


@jax.jit
def _sc_gather(idx_flat, x):
    mesh = plsc.VectorSubcoreMesh(core_axis_name="c", subcore_axis_name="s")
    f = pl.kernel(
        _gather_body,
        mesh=mesh,
        out_type=jax.ShapeDtypeStruct((TOTAL_ROWS, D_FEAT), jnp.float32),
        scratch_types=[
            pltpu.VMEM((MAX_CHUNKS_W * CHUNK,), jnp.int32),
            pltpu.VMEM((NBUF, CHUNK, D_FEAT), jnp.float32),
            pltpu.SemaphoreType.DMA((NBUF,)),
            pltpu.SemaphoreType.DMA((NBUF,)),
        ],
    )
    return f(idx_flat, x)


def kernel(path_index_without_target, x, att):
    del att
    idx = path_index_without_target.T.reshape(-1).astype(jnp.int32)
    idx = jnp.pad(idx, (0, IDX_PAD - TOTAL_ROWS))
    out = _sc_gather(idx, x.astype(jnp.float32))
    return out.reshape(PATH_LEN, NUM_PATHS, D_FEAT)


# speedup vs baseline: 1.0296x; 1.0015x over previous
import jax
import jax.numpy as jnp
from jax import lax
from jax.experimental import pallas as pl
from jax.experimental.pallas import tpu as pltpu
from jax.experimental.pallas import tpu_sc as plsc

N_NODES = 10000
D_FEAT = 256
NUM_PATHS = 50000
PATH_LEN = 4

TOTAL_ROWS = NUM_PATHS * PATH_LEN
CHUNK = 80
NUM_CHUNKS = TOTAL_ROWS // CHUNK
NUM_WORKERS = 32
BASE_CHUNKS = NUM_CHUNKS // NUM_WORKERS
EXTRA = NUM_CHUNKS - BASE_CHUNKS * NUM_WORKERS
MAX_CHUNKS_W = BASE_CHUNKS + 1
NBUF = 6
SDELAY = 3
STEPS = (MAX_CHUNKS_W + SDELAY + NBUF - 1) // NBUF
IDX_PAD = ((NUM_WORKERS - 1) * BASE_CHUNKS + EXTRA + MAX_CHUNKS_W) * CHUNK


def _gather_body(idx_hbm, x_hbm, out_hbm, idx_v, rows_v, gsem, ssem):
    nc = jnp.int32(2)
    wid = lax.axis_index("s") * nc + lax.axis_index("c")
    count = jnp.int32(BASE_CHUNKS) + jnp.where(wid < EXTRA, 1, 0).astype(jnp.int32)
    start = jnp.int32(BASE_CHUNKS) * wid + jnp.minimum(wid, jnp.int32(EXTRA))
    pltpu.sync_copy(
        idx_hbm.at[pl.ds(start * jnp.int32(CHUNK), MAX_CHUNKS_W * CHUNK)], idx_v)

    def gather_args(j, b):
        bi = jnp.int32(b)
        idx_c = idx_v.at[pl.ds(j * jnp.int32(CHUNK), CHUNK)]
        return x_hbm.at[idx_c], rows_v.at[bi], gsem.at[bi]

    def store_args(j, b):
        bi = jnp.int32(b)
        row0 = (start + j) * jnp.int32(CHUNK)
        return rows_v.at[bi], out_hbm.at[pl.ds(row0, CHUNK)], ssem.at[bi]

    def gather(j, b):
        pltpu.async_copy(*gather_args(j, b))

    def gather_wait(j, b):
        pltpu.make_async_copy(*gather_args(j, b)).wait()

    def store(j, b):
        pltpu.async_copy(*store_args(j, b))

    def store_wait(j, b):
        pltpu.make_async_copy(*store_args(j, b)).wait()

    for b in range(NBUF):
        gather(jnp.int32(b), b)

    @pl.loop(jnp.int32(0), jnp.int32(STEPS))
    def step(s):
        jbase = s * jnp.int32(NBUF)
        for b in range(NBUF):
            j = jbase + jnp.int32(b)

            @pl.when(j < count)
            def _():
                gather_wait(j, b % NBUF)
                store(j, b % NBUF)

            jp = j - jnp.int32(SDELAY)
            bp = (b - SDELAY) % NBUF
            jn = jp + jnp.int32(NBUF)

            @pl.when((jp >= 0) & (jp < count))
            def _():
                store_wait(jp, bp)

            @pl.when((jn >= jnp.int32(NBUF)) & (jn < count))
            def _():
                gather(jn, bp)


@jax.jit
def _sc_gather(idx_flat, x):
    mesh = plsc.VectorSubcoreMesh(core_axis_name="c", subcore_axis_name="s")
    f = pl.kernel(
        _gather_body,
        mesh=mesh,
        out_type=jax.ShapeDtypeStruct((TOTAL_ROWS, D_FEAT), jnp.float32),
        scratch_types=[
            pltpu.VMEM((MAX_CHUNKS_W * CHUNK,), jnp.int32),
            pltpu.VMEM((NBUF, CHUNK, D_FEAT), jnp.float32),
            pltpu.SemaphoreType.DMA((NBUF,)),
            pltpu.SemaphoreType.DMA((NBUF,)),
        ],
    )
    return f(idx_flat, x)


def kernel(path_index_without_target, x, att):
    del att
    idx = path_index_without_target.T.reshape(-1).astype(jnp.int32)
    idx = jnp.pad(idx, (0, IDX_PAD - TOTAL_ROWS))
    out = _sc_gather(idx, x.astype(jnp.float32))
    return out.reshape(PATH_LEN, NUM_PATHS, D_FEAT)
